# baseline (speedup 1.0000x reference)
import functools

import jax
import jax.numpy as jnp
from jax.experimental import pallas as pl
from jax.experimental.pallas import tpu as pltpu

_MT = 200


def _hi_dot(x, w):
    return jax.lax.dot_general(
        x, w, (((1,), (0,)), ((), ())),
        precision=jax.lax.Precision.HIGHEST,
        preferred_element_type=jnp.float32)


def _body(x_ref, w1_ref, b1_ref, w2_ref, b2_ref, adj_ref, o_ref,
          s1_ref, h2_ref, *, ni):
    g = pl.program_id(0)

    @pl.when(g == 0)
    def _():
        s1_ref[...] = _hi_dot(x_ref[...], w1_ref[...]).astype(jnp.bfloat16)

    @pl.when((g >= 1) & (g <= ni))
    def _():
        a = adj_ref[...].astype(jnp.bfloat16)
        acc = jnp.dot(a, s1_ref[...], preferred_element_type=jnp.float32)
        h = jnp.maximum(acc + b1_ref[...], 0.0)
        h2_ref[pl.ds((g - 1) * _MT, _MT), :] = (
            _hi_dot(h, w2_ref[...]).astype(jnp.bfloat16))

    @pl.when(g > ni)
    def _():
        a = adj_ref[...].astype(jnp.bfloat16)
        x = jnp.dot(a, h2_ref[...], preferred_element_type=jnp.float32)
        x = x + b2_ref[...]
        m = jnp.max(x, axis=1, keepdims=True)
        s = x - m
        o_ref[...] = s - jnp.log(jnp.sum(jnp.exp(s), axis=1, keepdims=True))


def kernel(feature, adj, W1, b1, W2, b2):
    n, d_in = feature.shape
    d_hid = W1.shape[1]
    d_out = W2.shape[1]
    ni = n // _MT

    def adj_idx(g):
        return (jnp.where(g == 0, 0, (g - 1) % ni), 0)

    def out_idx(g):
        return (jnp.where(g <= ni, 0, g - ni - 1), 0)

    return pl.pallas_call(
        functools.partial(_body, ni=ni),
        grid=(2 * ni + 1,),
        in_specs=[
            pl.BlockSpec((n, d_in), lambda g: (0, 0)),
            pl.BlockSpec((d_in, d_hid), lambda g: (0, 0)),
            pl.BlockSpec((1, d_hid), lambda g: (0, 0)),
            pl.BlockSpec((d_hid, d_out), lambda g: (0, 0)),
            pl.BlockSpec((1, d_out), lambda g: (0, 0)),
            pl.BlockSpec((_MT, n), adj_idx),
        ],
        out_specs=pl.BlockSpec((_MT, d_out), out_idx),
        out_shape=jax.ShapeDtypeStruct((n, d_out), jnp.float32),
        scratch_shapes=[
            pltpu.VMEM((n, d_hid), jnp.bfloat16),
            pltpu.VMEM((n, d_out), jnp.bfloat16),
        ],
        compiler_params=pltpu.CompilerParams(
            dimension_semantics=("arbitrary",)),
    )(feature, W1, b1.reshape(1, -1), W2, b2.reshape(1, -1), adj)


# speedup vs baseline: 1.0400x; 1.0400x over previous
import functools

import jax
import jax.numpy as jnp
from jax.experimental import pallas as pl
from jax.experimental.pallas import tpu as pltpu

_MT = 400


def _hi_dot(x, w):
    return jax.lax.dot_general(
        x, w, (((1,), (0,)), ((), ())),
        precision=jax.lax.Precision.HIGHEST,
        preferred_element_type=jnp.float32)


def _s1_body(x_ref, w1_ref, o_ref):
    o_ref[...] = _hi_dot(x_ref[...], w1_ref[...]).astype(jnp.bfloat16)


def _body(s1_ref, b1_ref, w2_ref, b2_ref, adj_ref, o_ref, h2_ref, *, ni):
    g = pl.program_id(0)
    a = adj_ref[...].astype(jnp.bfloat16)

    @pl.when(g < ni)
    def _():
        acc = jnp.dot(a, s1_ref[...], preferred_element_type=jnp.float32)
        h = jnp.maximum(acc + b1_ref[...], 0.0)
        h2_ref[pl.ds(g * _MT, _MT), :] = (
            _hi_dot(h, w2_ref[...]).astype(jnp.bfloat16))

    @pl.when(g >= ni)
    def _():
        x = jnp.dot(a, h2_ref[...], preferred_element_type=jnp.float32)
        x = x + b2_ref[...]
        m = jnp.max(x, axis=1, keepdims=True)
        s = x - m
        o_ref[...] = s - jnp.log(jnp.sum(jnp.exp(s), axis=1, keepdims=True))


def kernel(feature, adj, W1, b1, W2, b2):
    n, d_in = feature.shape
    d_hid = W1.shape[1]
    d_out = W2.shape[1]
    ni = n // _MT

    s1 = pl.pallas_call(
        _s1_body,
        out_shape=jax.ShapeDtypeStruct((n, d_hid), jnp.bfloat16),
    )(feature, W1)

    return pl.pallas_call(
        functools.partial(_body, ni=ni),
        grid=(2 * ni,),
        in_specs=[
            pl.BlockSpec((n, d_hid), lambda g: (0, 0)),
            pl.BlockSpec((1, d_hid), lambda g: (0, 0)),
            pl.BlockSpec((d_hid, d_out), lambda g: (0, 0)),
            pl.BlockSpec((1, d_out), lambda g: (0, 0)),
            pl.BlockSpec((_MT, n), lambda g: (g % ni, 0)),
        ],
        out_specs=pl.BlockSpec(
            (_MT, d_out), lambda g: (jnp.where(g < ni, 0, g - ni), 0)),
        out_shape=jax.ShapeDtypeStruct((n, d_out), jnp.float32),
        scratch_shapes=[
            pltpu.VMEM((n, d_out), jnp.bfloat16),
        ],
        compiler_params=pltpu.CompilerParams(
            dimension_semantics=("arbitrary",)),
    )(s1, b1.reshape(1, -1), W2, b2.reshape(1, -1), adj)
